# baseline (speedup 1.0000x reference)
import jax
import jax.numpy as jnp
from jax.experimental import pallas as pl

_BN = 2048


def _pfc_kernel(a_ref, w_ref, o_ref):
    a = a_ref[...].astype(jnp.bfloat16)
    w = w_ref[...].astype(jnp.bfloat16)
    o_ref[...] = jax.lax.dot_general(
        a, w,
        dimension_numbers=(((1,), (1,)), ((), ())),
        preferred_element_type=jnp.float32,
    )


def kernel(total_features, norm_weight):
    b, k = total_features.shape
    n = norm_weight.shape[0]
    grid = (pl.cdiv(n, _BN),)
    return pl.pallas_call(
        _pfc_kernel,
        grid=grid,
        in_specs=[
            pl.BlockSpec((b, k), lambda i: (0, 0)),
            pl.BlockSpec((_BN, k), lambda i: (i, 0)),
        ],
        out_specs=pl.BlockSpec((b, _BN), lambda i: (0, i)),
        out_shape=jax.ShapeDtypeStruct((b, n), jnp.float32),
    )(total_features, norm_weight)
